# baseline (speedup 1.0000x reference)
import jax
import jax.numpy as jnp
from jax.experimental import pallas as pl
from jax.experimental.pallas import tpu as pltpu

_CHUNKS = 8


def _copy_body(in_ref, out_ref, sems):
    rows = in_ref.shape[0] // _CHUNKS
    for i in range(_CHUNKS):
        src = in_ref.at[pl.ds(i * rows, rows)]
        dst = out_ref.at[pl.ds(i * rows, rows)]
        pltpu.make_async_copy(src, dst, sems.at[i]).start()
    for i in range(_CHUNKS):
        src = in_ref.at[pl.ds(i * rows, rows)]
        dst = out_ref.at[pl.ds(i * rows, rows)]
        pltpu.make_async_copy(src, dst, sems.at[i]).wait()


def kernel(input):
    b, c, h, w = input.shape
    flat = input.reshape(b * c, h * w)
    out = pl.pallas_call(
        _copy_body,
        out_shape=jax.ShapeDtypeStruct(flat.shape, flat.dtype),
        in_specs=[pl.BlockSpec(memory_space=pl.ANY)],
        out_specs=pl.BlockSpec(memory_space=pl.ANY),
        scratch_shapes=[pltpu.SemaphoreType.DMA((_CHUNKS,))],
    )(flat)
    return out.reshape(b, c, h, w)


# speedup vs baseline: 7.2230x; 7.2230x over previous
import jax
import jax.numpy as jnp
from jax.experimental import pallas as pl
from jax.experimental.pallas import tpu as pltpu

_BLOCK_ROWS = 128


def _copy_body(in_ref, out_ref):
    out_ref[...] = in_ref[...]


def kernel(input):
    b, c, h, w = input.shape
    rows, cols = b * c, h * w
    flat = input.reshape(rows, cols)
    grid = (rows // _BLOCK_ROWS,)
    out = pl.pallas_call(
        _copy_body,
        out_shape=jax.ShapeDtypeStruct(flat.shape, flat.dtype),
        grid=grid,
        in_specs=[pl.BlockSpec((_BLOCK_ROWS, cols), lambda i: (i, 0))],
        out_specs=pl.BlockSpec((_BLOCK_ROWS, cols), lambda i: (i, 0)),
    )(flat)
    return out.reshape(b, c, h, w)


# speedup vs baseline: 12.4118x; 1.7184x over previous
import jax
import jax.numpy as jnp
from jax.experimental import pallas as pl
from jax.experimental.pallas import tpu as pltpu

_BLOCK_C = 64


def _copy_body(in_ref, out_ref):
    out_ref[...] = in_ref[...]


def kernel(input):
    b, c, h, w = input.shape
    grid = (b, c // _BLOCK_C)
    return pl.pallas_call(
        _copy_body,
        out_shape=jax.ShapeDtypeStruct(input.shape, input.dtype),
        grid=grid,
        in_specs=[pl.BlockSpec((1, _BLOCK_C, h, w), lambda i, j: (i, j, 0, 0))],
        out_specs=pl.BlockSpec((1, _BLOCK_C, h, w), lambda i, j: (i, j, 0, 0)),
    )(input)
